# baseline (speedup 1.0000x reference)
import functools

import jax
import jax.numpy as jnp
from jax.experimental import pallas as pl
from jax.experimental.pallas import tpu as pltpu


def _rms(x, g, eps):
    ms = jnp.mean(jnp.square(x), axis=-1, keepdims=True)
    return x * jax.lax.rsqrt(ms + eps) * g


def _fused_kernel(
    lat_ref, emb_ref, mask_ref,
    g1_ref, wq_hbm, bq_ref, wk_hbm, bk_ref, wv_hbm, bv_ref, wo_hbm, bo_ref,
    g2_ref, w1_hbm, b1_ref, w2_hbm, b2_ref,
    out_ref,
    wq_s, wk_s, wv_s, wo_s, w1_s, w2_s, st_a, st_b, sem_a, sem_b,
    *, num_heads, q_scale,
):
    eps = jnp.float32(jnp.finfo(jnp.float32).eps)
    H = num_heads
    D = lat_ref.shape[-1]
    hd = D // H
    HID = w1_s.shape[-1]

    ch_a = st_a.shape[1]
    ch_b = st_b.shape[1]
    chunks_a = []
    group_of = {}
    for src, dst in ((wk_hbm, wk_s), (wv_hbm, wv_s), (wq_hbm, wq_s),
                     (wo_hbm, wo_s), (w2_hbm, w2_s)):
        group_of[id(dst)] = []
        for r in range(0, src.shape[0], ch_a):
            group_of[id(dst)].append(len(chunks_a))
            chunks_a.append((src, r, dst))
    chunks_b = [(w1_hbm, r, w1_s) for r in range(0, D, ch_b)]

    slots_a = st_a.shape[0]
    slots_b = st_b.shape[0]

    def copy_a(i):
        src, r, _ = chunks_a[i]
        return pltpu.make_async_copy(
            src.at[pl.ds(r, ch_a), :], st_a.at[i % slots_a], sem_a.at[i % slots_a])

    def copy_b(i):
        src, r, _ = chunks_b[i]
        return pltpu.make_async_copy(
            src.at[pl.ds(r, ch_b), :], st_b.at[i % slots_b], sem_b.at[i % slots_b])

    def drain_a(lo, hi, hi_start):
        for i in range(lo, hi):
            copy_a(i).wait()
            _, r, d = chunks_a[i]
            val = st_a[i % slots_a]
            if d is wq_s:
                val = val * jnp.float32(q_scale)
            d[pl.ds(r, ch_a), :] = val.astype(jnp.bfloat16)
            if i + slots_a < hi_start:
                copy_a(i + slots_a).start()

    def drain_b(lo, hi):
        for i in range(lo, hi):
            copy_b(i).wait()
            _, r, d = chunks_b[i]
            d[pl.ds(r, ch_b), :] = st_b[i % slots_b].astype(jnp.bfloat16)
            if i + slots_b < hi:
                copy_b(i + slots_b).start()

    n_a_head = len(chunks_a) - len(group_of[id(w2_s)])
    first_step = pl.program_id(0) == 0

    @pl.when(first_step)
    def _load_head():
        for i in range(min(slots_a, n_a_head)):
            copy_a(i).start()
        drain_a(0, n_a_head, n_a_head)
        for i in range(min(slots_b, len(chunks_b))):
            copy_b(i).start()
        for i in range(n_a_head, min(n_a_head + slots_a, len(chunks_a))):
            copy_a(i).start()

    x = lat_ref[...]
    xn = _rms(x, g1_ref[...], eps)
    xn_b = xn.astype(jnp.bfloat16)

    e = emb_ref[...].astype(jnp.bfloat16)
    kf = jnp.dot(e, wk_s[...], preferred_element_type=jnp.float32) + bk_ref[...]
    vf = jnp.dot(e, wv_s[...], preferred_element_type=jnp.float32) + bv_ref[...]
    kb = kf.astype(jnp.bfloat16)
    vb = vf.astype(jnp.bfloat16)

    q = (jnp.dot(xn_b, wq_s[...], preferred_element_type=jnp.float32)
         + bq_ref[...] * jnp.float32(q_scale))
    qb = q.astype(jnp.bfloat16)

    mask = mask_ref[pl.ds(pl.program_id(0), 1), :]
    bias = jnp.where(mask > 0, jnp.float32(0.0), jnp.float32(-1e30))

    ctx_t_parts = []
    for h in range(H):
        sl = slice(h * hd, (h + 1) * hd)
        s = jax.lax.dot_general(
            qb[:, sl], kb[:, sl],
            (((1,), (1,)), ((), ())),
            preferred_element_type=jnp.float32)
        s = s + bias
        m = jnp.max(s, axis=-1, keepdims=True)
        p = jnp.exp2(s - m)
        denom = jnp.sum(p, axis=-1, keepdims=True)
        recip = pl.reciprocal(denom, approx=True).reshape(1, -1)
        ctx_t = jax.lax.dot_general(
            vb[:, sl], p.astype(jnp.bfloat16),
            (((0,), (1,)), ((), ())),
            preferred_element_type=jnp.float32)
        ctx_t_parts.append((ctx_t * recip).astype(jnp.bfloat16))
    ctx_t = jnp.concatenate(ctx_t_parts, axis=0)

    attn = jax.lax.dot_general(
        ctx_t, wo_s[...],
        (((0,), (0,)), ((), ())),
        preferred_element_type=jnp.float32) + bo_ref[...]
    x1 = attn + xn

    x2 = _rms(x1, g2_ref[...], eps)

    @pl.when(first_step)
    def _load_w1():
        drain_b(0, len(chunks_b))

    h1 = jnp.dot(x2.astype(jnp.bfloat16), w1_s[...],
                 preferred_element_type=jnp.float32) + b1_ref[...]
    h1 = h1 * jax.nn.sigmoid(h1)

    @pl.when(first_step)
    def _load_w2():
        drain_a(n_a_head, len(chunks_a), len(chunks_a))

    ff = jnp.dot(h1.astype(jnp.bfloat16), w2_s[...],
                 preferred_element_type=jnp.float32) + b2_ref[...]

    out_ref[...] = (ff + x2).astype(out_ref.dtype)


def kernel(latents, embeddings, mask, g1, wq, bq, wk, bk, wv, bv, wo, bo,
           g2, w1, b1, w2, b2):
    B, L, D = latents.shape
    _, S, _ = embeddings.shape
    H = 8
    hd = D // H
    HID = w1.shape[-1]
    scale = 1.4426950408889634 / float(hd) ** 0.5

    if mask is None:
        mask = jnp.ones((B, S), dtype=jnp.float32)
    mask = mask.astype(jnp.float32)

    def row(v):
        return jnp.asarray(v).reshape(1, -1).astype(jnp.float32)

    f32 = jnp.float32
    params = [
        row(g1),
        wq.astype(f32), row(bq),
        wk.astype(f32), row(bk),
        wv.astype(f32), row(bv),
        wo.astype(f32), row(bo),
        row(g2),
        w1.astype(f32), row(b1),
        w2.astype(f32), row(b2),
    ]
    hbm_idx = {1, 3, 5, 7, 10, 12}

    buffered = getattr(pl, "Buffered", None)

    def build(single_buffer_weights):
        wkw = {"pipeline_mode": buffered(1)} if single_buffer_weights else {}

        def spec_for(i, arr):
            if i in hbm_idx:
                return pl.BlockSpec(memory_space=pl.ANY)
            nd = arr.ndim
            return pl.BlockSpec(arr.shape, lambda b, _nd=nd: (0,) * _nd, **wkw)

        in_specs = [
            pl.BlockSpec((None, L, D), lambda b: (b, 0, 0)),
            pl.BlockSpec((None, S, D), lambda b: (b, 0, 0)),
            pl.BlockSpec((B, S), lambda b: (0, 0), **wkw),
        ] + [spec_for(i, w) for i, w in enumerate(params)]

        return pl.pallas_call(
            functools.partial(_fused_kernel, num_heads=H, q_scale=scale),
            out_shape=jax.ShapeDtypeStruct((B, L, D), latents.dtype),
            grid=(B,),
            in_specs=in_specs,
            out_specs=pl.BlockSpec((None, L, D), lambda b: (b, 0, 0)),
            scratch_shapes=[
                pltpu.VMEM((D, D), jnp.bfloat16),
                pltpu.VMEM((D, D), jnp.bfloat16),
                pltpu.VMEM((D, D), jnp.bfloat16),
                pltpu.VMEM((D, D), jnp.bfloat16),
                pltpu.VMEM((D, HID), jnp.bfloat16),
                pltpu.VMEM((HID, D), jnp.bfloat16),
                pltpu.VMEM((3, min(512, D), D), jnp.float32),
                pltpu.VMEM((2, min(128, D), HID), jnp.float32),
                pltpu.SemaphoreType.DMA((3,)),
                pltpu.SemaphoreType.DMA((2,)),
            ],
            compiler_params=pltpu.CompilerParams(
                dimension_semantics=("arbitrary",),
                vmem_limit_bytes=(64 * 1024 * 1024 * 15) // 16,
            ),
        )

    args = (latents.astype(jnp.float32), embeddings.astype(jnp.float32),
            mask, *params)
    if buffered is not None:
        try:
            return build(True)(*args)
        except Exception:
            return build(False)(*args)
    return build(False)(*args)


# speedup vs baseline: 1.0073x; 1.0073x over previous
import functools

import jax
import jax.numpy as jnp
from jax.experimental import pallas as pl
from jax.experimental.pallas import tpu as pltpu


def _rms(x, g, eps):
    ms = jnp.mean(jnp.square(x), axis=-1, keepdims=True)
    return x * jax.lax.rsqrt(ms + eps) * g


def _fused_kernel(
    lat_ref, emb_ref, mask_ref,
    g1_ref, wq_hbm, bq_ref, wk_hbm, bk_ref, wv_hbm, bv_ref, wo_hbm, bo_ref,
    g2_ref, w1_hbm, b1_ref, w2_hbm, b2_ref,
    out_ref,
    wq_s, wk_s, wv_s, wo_s, w1_s, w2_s, st_a, st_b, sem_a, sem_b,
    *, num_heads, q_scale,
):
    eps = jnp.float32(jnp.finfo(jnp.float32).eps)
    H = num_heads
    D = lat_ref.shape[-1]
    hd = D // H
    HID = w1_s.shape[-1]

    ch_a = st_a.shape[1]
    ch_b = st_b.shape[1]
    chunks_a = []
    group_of = {}
    for src, dst in ((wk_hbm, wk_s), (wv_hbm, wv_s), (wq_hbm, wq_s),
                     (wo_hbm, wo_s), (w2_hbm, w2_s)):
        group_of[id(dst)] = []
        for r in range(0, src.shape[0], ch_a):
            group_of[id(dst)].append(len(chunks_a))
            chunks_a.append((src, r, dst))
    chunks_b = [(w1_hbm, r, w1_s) for r in range(0, D, ch_b)]

    slots_a = st_a.shape[0]
    slots_b = st_b.shape[0]

    def copy_a(i):
        src, r, _ = chunks_a[i]
        return pltpu.make_async_copy(
            src.at[pl.ds(r, ch_a), :], st_a.at[i % slots_a], sem_a.at[i % slots_a])

    def copy_b(i):
        src, r, _ = chunks_b[i]
        return pltpu.make_async_copy(
            src.at[pl.ds(r, ch_b), :], st_b.at[i % slots_b], sem_b.at[i % slots_b])

    def drain_a(lo, hi, hi_start):
        for i in range(lo, hi):
            copy_a(i).wait()
            _, r, d = chunks_a[i]
            val = st_a[i % slots_a]
            if d is wq_s:
                val = val * jnp.float32(q_scale)
            d[pl.ds(r, ch_a), :] = val.astype(jnp.bfloat16)
            if i + slots_a < hi_start:
                copy_a(i + slots_a).start()

    def drain_b(lo, hi):
        for i in range(lo, hi):
            copy_b(i).wait()
            _, r, d = chunks_b[i]
            d[pl.ds(r, ch_b), :] = st_b[i % slots_b].astype(jnp.bfloat16)
            if i + slots_b < hi:
                copy_b(i + slots_b).start()

    n_a_head = len(chunks_a) - len(group_of[id(w2_s)])
    first_step = pl.program_id(0) == 0

    @pl.when(first_step)
    def _load_head():
        for i in range(min(slots_a, n_a_head)):
            copy_a(i).start()
        drain_a(0, n_a_head, n_a_head)
        for i in range(min(slots_b, len(chunks_b))):
            copy_b(i).start()
        for i in range(n_a_head, min(n_a_head + slots_a, len(chunks_a))):
            copy_a(i).start()

    x = lat_ref[...]
    xn = _rms(x, g1_ref[...], eps)
    xn_b = xn.astype(jnp.bfloat16)

    e = emb_ref[...].astype(jnp.bfloat16)
    kf = jnp.dot(e, wk_s[...], preferred_element_type=jnp.float32) + bk_ref[...]
    vf = jnp.dot(e, wv_s[...], preferred_element_type=jnp.float32) + bv_ref[...]
    kb = kf.astype(jnp.bfloat16)
    vb = vf.astype(jnp.bfloat16)

    q = (jnp.dot(xn_b, wq_s[...], preferred_element_type=jnp.float32)
         + bq_ref[...] * jnp.float32(q_scale))
    qb = q.astype(jnp.bfloat16)

    mask = mask_ref[pl.ds(pl.program_id(0), 1), :]
    bias = jnp.where(mask > 0, jnp.float32(0.0), jnp.float32(-1e30))

    ctx_t_parts = []
    for h in range(H):
        sl = slice(h * hd, (h + 1) * hd)
        s = jax.lax.dot_general(
            qb[:, sl], kb[:, sl],
            (((1,), (1,)), ((), ())),
            preferred_element_type=jnp.float32)
        s = s + bias
        m = jnp.max(s, axis=-1, keepdims=True)
        p = jnp.exp(s - m)
        denom = jnp.sum(p, axis=-1, keepdims=True)
        recip = pl.reciprocal(denom, approx=True).reshape(1, -1)
        ctx_t = jax.lax.dot_general(
            vb[:, sl], p.astype(jnp.bfloat16),
            (((0,), (1,)), ((), ())),
            preferred_element_type=jnp.float32)
        ctx_t_parts.append((ctx_t * recip).astype(jnp.bfloat16))
    ctx_t = jnp.concatenate(ctx_t_parts, axis=0)

    attn = jax.lax.dot_general(
        ctx_t, wo_s[...],
        (((0,), (0,)), ((), ())),
        preferred_element_type=jnp.float32) + bo_ref[...]
    x1 = attn + xn

    x2 = _rms(x1, g2_ref[...], eps)

    @pl.when(first_step)
    def _load_w1():
        drain_b(0, len(chunks_b))

    h1 = jnp.dot(x2.astype(jnp.bfloat16), w1_s[...],
                 preferred_element_type=jnp.float32) + b1_ref[...]
    h1 = h1 * jax.nn.sigmoid(h1)

    @pl.when(first_step)
    def _load_w2():
        drain_a(n_a_head, len(chunks_a), len(chunks_a))

    ff = jnp.dot(h1.astype(jnp.bfloat16), w2_s[...],
                 preferred_element_type=jnp.float32) + b2_ref[...]

    out_ref[...] = (ff + x2).astype(out_ref.dtype)


def kernel(latents, embeddings, mask, g1, wq, bq, wk, bk, wv, bv, wo, bo,
           g2, w1, b1, w2, b2):
    B, L, D = latents.shape
    _, S, _ = embeddings.shape
    H = 8
    hd = D // H
    HID = w1.shape[-1]
    scale = 1.0 / float(hd) ** 0.5

    if mask is None:
        mask = jnp.ones((B, S), dtype=jnp.float32)
    mask = mask.astype(jnp.float32)

    def row(v):
        return jnp.asarray(v).reshape(1, -1).astype(jnp.float32)

    f32 = jnp.float32
    params = [
        row(g1),
        wq.astype(f32), row(bq),
        wk.astype(f32), row(bk),
        wv.astype(f32), row(bv),
        wo.astype(f32), row(bo),
        row(g2),
        w1.astype(f32), row(b1),
        w2.astype(f32), row(b2),
    ]
    hbm_idx = {1, 3, 5, 7, 10, 12}

    buffered = getattr(pl, "Buffered", None)

    def build(single_buffer_weights):
        wkw = {"pipeline_mode": buffered(1)} if single_buffer_weights else {}

        def spec_for(i, arr):
            if i in hbm_idx:
                return pl.BlockSpec(memory_space=pl.ANY)
            nd = arr.ndim
            return pl.BlockSpec(arr.shape, lambda b, _nd=nd: (0,) * _nd, **wkw)

        in_specs = [
            pl.BlockSpec((None, L, D), lambda b: (b, 0, 0)),
            pl.BlockSpec((None, S, D), lambda b: (b, 0, 0)),
            pl.BlockSpec((B, S), lambda b: (0, 0), **wkw),
        ] + [spec_for(i, w) for i, w in enumerate(params)]

        return pl.pallas_call(
            functools.partial(_fused_kernel, num_heads=H, q_scale=scale),
            out_shape=jax.ShapeDtypeStruct((B, L, D), latents.dtype),
            grid=(B,),
            in_specs=in_specs,
            out_specs=pl.BlockSpec((None, L, D), lambda b: (b, 0, 0)),
            scratch_shapes=[
                pltpu.VMEM((D, D), jnp.bfloat16),
                pltpu.VMEM((D, D), jnp.bfloat16),
                pltpu.VMEM((D, D), jnp.bfloat16),
                pltpu.VMEM((D, D), jnp.bfloat16),
                pltpu.VMEM((D, HID), jnp.bfloat16),
                pltpu.VMEM((HID, D), jnp.bfloat16),
                pltpu.VMEM((3, min(512, D), D), jnp.float32),
                pltpu.VMEM((2, min(128, D), HID), jnp.float32),
                pltpu.SemaphoreType.DMA((3,)),
                pltpu.SemaphoreType.DMA((2,)),
            ],
            compiler_params=pltpu.CompilerParams(
                dimension_semantics=("arbitrary",),
                vmem_limit_bytes=(64 * 1024 * 1024 * 15) // 16,
            ),
        )

    args = (latents.astype(jnp.float32), embeddings.astype(jnp.float32),
            mask, *params)
    if buffered is not None:
        try:
            return build(True)(*args)
        except Exception:
            return build(False)(*args)
    return build(False)(*args)


# speedup vs baseline: 1.0225x; 1.0151x over previous
import functools

import jax
import jax.numpy as jnp
from jax.experimental import pallas as pl
from jax.experimental.pallas import tpu as pltpu


def _rms(x, g, eps):
    ms = jnp.mean(jnp.square(x), axis=-1, keepdims=True)
    return x * jax.lax.rsqrt(ms + eps) * g


def _fused_kernel(
    lat_ref, emb_ref, mask_ref,
    g1_ref, wq_hbm, bq_ref, wk_hbm, bk_ref, wv_hbm, bv_ref, wo_hbm, bo_ref,
    g2_ref, w1_hbm, b1_ref, w2_hbm, b2_ref,
    out_ref,
    wq_s, wk_s, wv_s, wo_s, w1_s, w2_s, st_a, st_b, sem_a, sem_b,
    *, num_heads, q_scale,
):
    eps = jnp.float32(jnp.finfo(jnp.float32).eps)
    H = num_heads
    D = lat_ref.shape[-1]
    hd = D // H

    ch_a = st_a.shape[1]
    ch_b = st_b.shape[1]
    chunks_a = []
    group_of = {}
    for src, dst in ((wk_hbm, wk_s), (wv_hbm, wv_s), (wq_hbm, wq_s),
                     (wo_hbm, wo_s), (w2_hbm, w2_s)):
        group_of[id(dst)] = []
        for r in range(0, src.shape[0], ch_a):
            group_of[id(dst)].append(len(chunks_a))
            chunks_a.append((src, r, dst))
    chunks_b = [(w1_hbm, r, w1_s) for r in range(0, D, ch_b)]

    slots_a = st_a.shape[0]
    slots_b = st_b.shape[0]

    def copy_a(i):
        src, r, _ = chunks_a[i]
        return pltpu.make_async_copy(
            src.at[pl.ds(r, ch_a), :], st_a.at[i % slots_a], sem_a.at[i % slots_a])

    def copy_b(i):
        src, r, _ = chunks_b[i]
        return pltpu.make_async_copy(
            src.at[pl.ds(r, ch_b), :], st_b.at[i % slots_b], sem_b.at[i % slots_b])

    def drain_a(lo, hi, hi_start):
        for i in range(lo, hi):
            copy_a(i).wait()
            _, r, d = chunks_a[i]
            val = st_a[i % slots_a]
            if d is wq_s:
                val = val * jnp.float32(q_scale)
            d[pl.ds(r, ch_a), :] = val.astype(jnp.bfloat16)
            if i + slots_a < hi_start:
                copy_a(i + slots_a).start()

    def drain_b(lo, hi):
        for i in range(lo, hi):
            copy_b(i).wait()
            _, r, d = chunks_b[i]
            d[pl.ds(r, ch_b), :] = st_b[i % slots_b].astype(jnp.bfloat16)
            if i + slots_b < hi:
                copy_b(i + slots_b).start()

    n_a_head = len(chunks_a) - len(group_of[id(w2_s)])
    first_step = pl.program_id(0) == 0

    @pl.when(first_step)
    def _load_head():
        for i in range(min(slots_a, n_a_head)):
            copy_a(i).start()
        drain_a(0, n_a_head, n_a_head)
        for i in range(min(slots_b, len(chunks_b))):
            copy_b(i).start()
        for i in range(n_a_head, min(n_a_head + slots_a, len(chunks_a))):
            copy_a(i).start()

    x = lat_ref[...]
    xn = _rms(x, g1_ref[...], eps)
    xn_b = xn.astype(jnp.bfloat16)

    e = emb_ref[...].astype(jnp.bfloat16)
    kf = jnp.dot(e, wk_s[...], preferred_element_type=jnp.float32) + bk_ref[...]
    vf = jnp.dot(e, wv_s[...], preferred_element_type=jnp.float32) + bv_ref[...]
    kb = kf.astype(jnp.bfloat16)
    vb = vf.astype(jnp.bfloat16)

    q = (jnp.dot(xn_b, wq_s[...], preferred_element_type=jnp.float32)
         + bq_ref[...] * jnp.float32(q_scale))
    qb = q.astype(jnp.bfloat16)

    mask = mask_ref[pl.ds(pl.program_id(0), 1), :]
    bias = jnp.where(mask > 0, jnp.float32(0.0), jnp.float32(-1e30))

    ctx_t_parts = []
    for h in range(H):
        sl = slice(h * hd, (h + 1) * hd)
        s = jax.lax.dot_general(
            qb[:, sl], kb[:, sl],
            (((1,), (1,)), ((), ())),
            preferred_element_type=jnp.float32)
        s = s + bias
        m = jnp.max(s, axis=-1, keepdims=True)
        p = jnp.exp(s - m)
        denom = jnp.sum(p, axis=-1, keepdims=True)
        recip = pl.reciprocal(denom, approx=True).reshape(1, -1)
        ctx_t = jax.lax.dot_general(
            vb[:, sl], p.astype(jnp.bfloat16),
            (((0,), (1,)), ((), ())),
            preferred_element_type=jnp.float32)
        ctx_t_parts.append((ctx_t * recip).astype(jnp.bfloat16))
    ctx_t = jnp.concatenate(ctx_t_parts, axis=0)

    attn = jax.lax.dot_general(
        ctx_t, wo_s[...],
        (((0,), (0,)), ((), ())),
        preferred_element_type=jnp.float32) + bo_ref[...]
    x1 = attn + xn

    x2 = _rms(x1, g2_ref[...], eps)

    @pl.when(first_step)
    def _load_w1():
        drain_b(0, len(chunks_b))

    h1 = jnp.dot(x2.astype(jnp.bfloat16), w1_s[...],
                 preferred_element_type=jnp.float32) + b1_ref[...]
    h1 = h1 * jax.nn.sigmoid(h1)

    @pl.when(first_step)
    def _load_w2():
        drain_a(n_a_head, len(chunks_a), len(chunks_a))

    ff = jnp.dot(h1.astype(jnp.bfloat16), w2_s[...],
                 preferred_element_type=jnp.float32) + b2_ref[...]

    out_ref[...] = (ff + x2).astype(out_ref.dtype)


def kernel(latents, embeddings, mask, g1, wq, bq, wk, bk, wv, bv, wo, bo,
           g2, w1, b1, w2, b2):
    B, L, D = latents.shape
    _, S, _ = embeddings.shape
    H = 8
    hd = D // H
    HID = w1.shape[-1]
    scale = 1.0 / float(hd) ** 0.5

    if mask is None:
        mask = jnp.ones((B, S), dtype=jnp.float32)
    mask = mask.astype(jnp.float32)

    def row(v):
        return jnp.asarray(v).reshape(1, -1).astype(jnp.float32)

    f32 = jnp.float32
    params = [
        row(g1),
        wq.astype(f32), row(bq),
        wk.astype(f32), row(bk),
        wv.astype(f32), row(bv),
        wo.astype(f32), row(bo),
        row(g2),
        w1.astype(f32), row(b1),
        w2.astype(f32), row(b2),
    ]
    hbm_idx = {1, 3, 5, 7, 10, 12}

    buffered = getattr(pl, "Buffered", None)

    def build(single_buffer_weights):
        wkw = {"pipeline_mode": buffered(1)} if single_buffer_weights else {}

        def spec_for(i, arr):
            if i in hbm_idx:
                return pl.BlockSpec(memory_space=pl.ANY)
            nd = arr.ndim
            return pl.BlockSpec(arr.shape, lambda b, _nd=nd: (0,) * _nd, **wkw)

        in_specs = [
            pl.BlockSpec((None, L, D), lambda b: (b, 0, 0)),
            pl.BlockSpec((None, S, D), lambda b: (b, 0, 0)),
            pl.BlockSpec((B, S), lambda b: (0, 0), **wkw),
        ] + [spec_for(i, w) for i, w in enumerate(params)]

        return pl.pallas_call(
            functools.partial(_fused_kernel, num_heads=H, q_scale=scale),
            out_shape=jax.ShapeDtypeStruct((B, L, D), latents.dtype),
            grid=(B,),
            in_specs=in_specs,
            out_specs=pl.BlockSpec((None, L, D), lambda b: (b, 0, 0)),
            scratch_shapes=[
                pltpu.VMEM((D, D), jnp.bfloat16),
                pltpu.VMEM((D, D), jnp.bfloat16),
                pltpu.VMEM((D, D), jnp.bfloat16),
                pltpu.VMEM((D, D), jnp.bfloat16),
                pltpu.VMEM((D, HID), jnp.bfloat16),
                pltpu.VMEM((HID, D), jnp.bfloat16),
                pltpu.VMEM((3, min(512, D), D), jnp.float32),
                pltpu.VMEM((3, min(128, D), HID), jnp.float32),
                pltpu.SemaphoreType.DMA((3,)),
                pltpu.SemaphoreType.DMA((3,)),
            ],
            compiler_params=pltpu.CompilerParams(
                dimension_semantics=("arbitrary",),
                vmem_limit_bytes=(64 * 1024 * 1024 * 15) // 16,
            ),
        )

    args = (latents.astype(jnp.float32), embeddings.astype(jnp.float32),
            mask, *params)
    if buffered is not None:
        try:
            return build(True)(*args)
        except Exception:
            return build(False)(*args)
    return build(False)(*args)


# speedup vs baseline: 1.0307x; 1.0080x over previous
import functools

import jax
import jax.numpy as jnp
from jax.experimental import pallas as pl
from jax.experimental.pallas import tpu as pltpu


def _rms(x, g, eps):
    ms = jnp.mean(jnp.square(x), axis=-1, keepdims=True)
    return x * jax.lax.rsqrt(ms + eps) * g


def _fused_kernel(
    lat_ref, emb_ref, mask_ref,
    g1_ref, wq_hbm, bq_ref, wk_hbm, bk_ref, wv_hbm, bv_ref, wo_hbm, bo_ref,
    g2_ref, w1_hbm, b1_ref, w2_hbm, b2_ref,
    out_ref,
    wq_s, wk_s, wv_s, wo_s, w1_s, w2_s, st_a, st_b, sem_a, sem_b,
    *, num_heads, q_scale,
):
    eps = jnp.float32(jnp.finfo(jnp.float32).eps)
    H = num_heads
    D = lat_ref.shape[-1]
    hd = D // H

    ch_a = st_a.shape[1]
    ch_b = st_b.shape[1]
    chunks_a = []
    group_of = {}
    for src, dst in ((wk_hbm, wk_s), (wv_hbm, wv_s), (wq_hbm, wq_s),
                     (wo_hbm, wo_s), (w2_hbm, w2_s)):
        group_of[id(dst)] = []
        for r in range(0, src.shape[0], ch_a):
            group_of[id(dst)].append(len(chunks_a))
            chunks_a.append((src, r, dst))
    chunks_b = [(w1_hbm, r, w1_s) for r in range(0, D, ch_b)]

    slots_a = st_a.shape[0]
    slots_b = st_b.shape[0]

    def copy_a(i):
        src, r, _ = chunks_a[i]
        return pltpu.make_async_copy(
            src.at[pl.ds(r, ch_a), :], st_a.at[i % slots_a], sem_a.at[i % slots_a])

    def copy_b(i):
        src, r, _ = chunks_b[i]
        return pltpu.make_async_copy(
            src.at[pl.ds(r, ch_b), :], st_b.at[i % slots_b], sem_b.at[i % slots_b])

    def drain_a(lo, hi, hi_start):
        for i in range(lo, hi):
            copy_a(i).wait()
            _, r, d = chunks_a[i]
            val = st_a[i % slots_a]
            if d is wq_s:
                val = val * jnp.float32(q_scale)
            d[pl.ds(r, ch_a), :] = val.astype(jnp.bfloat16)
            if i + slots_a < hi_start:
                copy_a(i + slots_a).start()

    def drain_b(lo, hi):
        for i in range(lo, hi):
            copy_b(i).wait()
            _, r, d = chunks_b[i]
            d[pl.ds(r, ch_b), :] = st_b[i % slots_b].astype(jnp.bfloat16)
            if i + slots_b < hi:
                copy_b(i + slots_b).start()

    n_a_head = len(chunks_a) - len(group_of[id(w2_s)])
    first_step = pl.program_id(0) == 0

    @pl.when(first_step)
    def _load_head():
        for i in range(min(slots_a, n_a_head)):
            copy_a(i).start()
        drain_a(0, n_a_head, n_a_head)
        for i in range(min(slots_b, len(chunks_b))):
            copy_b(i).start()
        for i in range(n_a_head, min(n_a_head + slots_a, len(chunks_a))):
            copy_a(i).start()

    x = lat_ref[...]
    xn = _rms(x, g1_ref[...], eps)
    xn_b = xn.astype(jnp.bfloat16)

    e = emb_ref[...].astype(jnp.bfloat16)
    kf = jnp.dot(e, wk_s[...], preferred_element_type=jnp.float32) + bk_ref[...]
    vf = jnp.dot(e, wv_s[...], preferred_element_type=jnp.float32) + bv_ref[...]
    kb = kf.astype(jnp.bfloat16)
    vb = vf.astype(jnp.bfloat16)

    q = (jnp.dot(xn_b, wq_s[...], preferred_element_type=jnp.float32)
         + bq_ref[...] * jnp.float32(q_scale))
    qb = q.astype(jnp.bfloat16)

    mask = mask_ref[pl.ds(pl.program_id(0), 1), :]
    bias = jnp.where(mask > 0, jnp.float32(0.0), jnp.float32(-1e30))

    ctx_t_parts = []
    for h in range(H):
        sl = slice(h * hd, (h + 1) * hd)
        s = jax.lax.dot_general(
            qb[:, sl], kb[:, sl],
            (((1,), (1,)), ((), ())),
            preferred_element_type=jnp.float32)
        s = s + bias
        m = jnp.max(s, axis=-1, keepdims=True)
        p = jnp.exp(s - m)
        denom = jnp.sum(p, axis=-1, keepdims=True)
        recip = pl.reciprocal(denom, approx=True).reshape(1, -1)
        ctx_t = jax.lax.dot_general(
            vb[:, sl], p.astype(jnp.bfloat16),
            (((0,), (1,)), ((), ())),
            preferred_element_type=jnp.float32)
        ctx_t_parts.append((ctx_t * recip).astype(jnp.bfloat16))
    ctx_t = jnp.concatenate(ctx_t_parts, axis=0)

    attn = jax.lax.dot_general(
        ctx_t, wo_s[...],
        (((0,), (0,)), ((), ())),
        preferred_element_type=jnp.float32) + bo_ref[...]
    x1 = attn + xn

    x2 = _rms(x1, g2_ref[...], eps)

    @pl.when(first_step)
    def _load_w1():
        drain_b(0, len(chunks_b))

    h1 = jnp.dot(x2.astype(jnp.bfloat16), w1_s[...],
                 preferred_element_type=jnp.float32) + b1_ref[...]
    h1 = h1 * jax.nn.sigmoid(h1)

    @pl.when(first_step)
    def _load_w2():
        drain_a(n_a_head, len(chunks_a), len(chunks_a))

    ff = jnp.dot(h1.astype(jnp.bfloat16), w2_s[...],
                 preferred_element_type=jnp.float32) + b2_ref[...]

    out_ref[...] = (ff + x2).astype(out_ref.dtype)


def kernel(latents, embeddings, mask, g1, wq, bq, wk, bk, wv, bv, wo, bo,
           g2, w1, b1, w2, b2):
    B, L, D = latents.shape
    _, S, _ = embeddings.shape
    H = 8
    hd = D // H
    HID = w1.shape[-1]
    scale = 1.0 / float(hd) ** 0.5

    if mask is None:
        mask = jnp.ones((B, S), dtype=jnp.float32)
    mask = mask.astype(jnp.float32)

    def row(v):
        return jnp.asarray(v).reshape(1, -1).astype(jnp.float32)

    f32 = jnp.float32
    params = [
        row(g1),
        wq.astype(f32), row(bq),
        wk.astype(f32), row(bk),
        wv.astype(f32), row(bv),
        wo.astype(f32), row(bo),
        row(g2),
        w1.astype(f32), row(b1),
        w2.astype(f32), row(b2),
    ]
    hbm_idx = {1, 3, 5, 7, 10, 12}

    buffered = getattr(pl, "Buffered", None)

    def build(single_buffer_weights):
        wkw = {"pipeline_mode": buffered(1)} if single_buffer_weights else {}

        def spec_for(i, arr):
            if i in hbm_idx:
                return pl.BlockSpec(memory_space=pl.ANY)
            nd = arr.ndim
            return pl.BlockSpec(arr.shape, lambda b, _nd=nd: (0,) * _nd, **wkw)

        in_specs = [
            pl.BlockSpec((None, L, D), lambda b: (b, 0, 0)),
            pl.BlockSpec((None, S, D), lambda b: (b, 0, 0)),
            pl.BlockSpec((B, S), lambda b: (0, 0), **wkw),
        ] + [spec_for(i, w) for i, w in enumerate(params)]

        return pl.pallas_call(
            functools.partial(_fused_kernel, num_heads=H, q_scale=scale),
            out_shape=jax.ShapeDtypeStruct((B, L, D), latents.dtype),
            grid=(B,),
            in_specs=in_specs,
            out_specs=pl.BlockSpec((None, L, D), lambda b: (b, 0, 0)),
            scratch_shapes=[
                pltpu.VMEM((D, D), jnp.bfloat16),
                pltpu.VMEM((D, D), jnp.bfloat16),
                pltpu.VMEM((D, D), jnp.bfloat16),
                pltpu.VMEM((D, D), jnp.bfloat16),
                pltpu.VMEM((D, HID), jnp.bfloat16),
                pltpu.VMEM((HID, D), jnp.bfloat16),
                pltpu.VMEM((4, min(512, D), D), jnp.float32),
                pltpu.VMEM((3, min(128, D), HID), jnp.float32),
                pltpu.SemaphoreType.DMA((4,)),
                pltpu.SemaphoreType.DMA((3,)),
            ],
            compiler_params=pltpu.CompilerParams(
                dimension_semantics=("arbitrary",),
                vmem_limit_bytes=(64 * 1024 * 1024 * 15) // 16,
            ),
        )

    args = (latents.astype(jnp.float32), embeddings.astype(jnp.float32),
            mask, *params)
    if buffered is not None:
        try:
            return build(True)(*args)
        except Exception:
            return build(False)(*args)
    return build(False)(*args)


# speedup vs baseline: 1.0585x; 1.0269x over previous
import functools

import jax
import jax.numpy as jnp
from jax.experimental import pallas as pl
from jax.experimental.pallas import tpu as pltpu


def _rms(x, g, eps):
    ms = jnp.mean(jnp.square(x), axis=-1, keepdims=True)
    return x * jax.lax.rsqrt(ms + eps) * g


def _fused_kernel(
    lat_ref, emb_ref, mask_ref,
    g1_ref, wq_hbm, bq_ref, wk_hbm, bk_ref, wv_hbm, bv_ref, wo_hbm, bo_ref,
    g2_ref, w1_hbm, b1_ref, w2_hbm, b2_ref,
    out_ref,
    wq_s, wk_s, wv_s, wo_s, w1_s, w2_s, st_a, st_b, sem_a, sem_b,
    *, num_heads, q_scale,
):
    eps = jnp.float32(jnp.finfo(jnp.float32).eps)
    H = num_heads
    D = lat_ref.shape[-1]
    hd = D // H

    ch_a = st_a.shape[1]
    ch_b = st_b.shape[1]
    chunks_a = []
    group_of = {}
    for src, dst in ((wk_hbm, wk_s), (wv_hbm, wv_s), (wq_hbm, wq_s),
                     (wo_hbm, wo_s), (w2_hbm, w2_s)):
        group_of[id(dst)] = []
        for r in range(0, src.shape[0], ch_a):
            group_of[id(dst)].append(len(chunks_a))
            chunks_a.append((src, r, dst))
    chunks_b = [(w1_hbm, r, w1_s) for r in range(0, D, ch_b)]

    slots_a = st_a.shape[0]
    slots_b = st_b.shape[0]

    def copy_a(i):
        src, r, _ = chunks_a[i]
        return pltpu.make_async_copy(
            src.at[pl.ds(r, ch_a), :], st_a.at[i % slots_a], sem_a.at[i % slots_a])

    def copy_b(i):
        src, r, _ = chunks_b[i]
        return pltpu.make_async_copy(
            src.at[pl.ds(r, ch_b), :], st_b.at[i % slots_b], sem_b.at[i % slots_b])

    def drain_a(lo, hi, hi_start):
        for i in range(lo, hi):
            copy_a(i).wait()
            _, r, d = chunks_a[i]
            val = st_a[i % slots_a]
            if d is wq_s:
                val = val * jnp.float32(q_scale)
            d[pl.ds(r, ch_a), :] = val.astype(jnp.bfloat16)
            if i + slots_a < hi_start:
                copy_a(i + slots_a).start()

    def drain_b(lo, hi):
        for i in range(lo, hi):
            copy_b(i).wait()
            _, r, d = chunks_b[i]
            d[pl.ds(r, ch_b), :] = st_b[i % slots_b].astype(jnp.bfloat16)
            if i + slots_b < hi:
                copy_b(i + slots_b).start()

    n_a_head = len(chunks_a) - len(group_of[id(w2_s)])
    first_step = pl.program_id(0) == 0

    @pl.when(first_step)
    def _load_head():
        for i in range(min(slots_a, n_a_head)):
            copy_a(i).start()
        drain_a(0, n_a_head, n_a_head)
        for i in range(min(slots_b, len(chunks_b))):
            copy_b(i).start()
        for i in range(n_a_head, min(n_a_head + slots_a, len(chunks_a))):
            copy_a(i).start()

    x = lat_ref[...]
    xn = _rms(x, g1_ref[...], eps)
    xn_b = xn.astype(jnp.bfloat16)

    e = emb_ref[...].astype(jnp.bfloat16)
    kf = jnp.dot(e, wk_s[...], preferred_element_type=jnp.float32) + bk_ref[...]
    vf = jnp.dot(e, wv_s[...], preferred_element_type=jnp.float32) + bv_ref[...]
    kb = kf.astype(jnp.bfloat16)
    vb = vf.astype(jnp.bfloat16)

    q = (jnp.dot(xn_b, wq_s[...], preferred_element_type=jnp.float32)
         + bq_ref[...] * jnp.float32(q_scale))
    qb = q.astype(jnp.bfloat16)

    mask = mask_ref[pl.ds(pl.program_id(0), 1), :]
    bias = jnp.where(mask > 0, jnp.float32(0.0), jnp.float32(-1e30))

    ctx_t_parts = []
    for h in range(H):
        sl = slice(h * hd, (h + 1) * hd)
        s = jax.lax.dot_general(
            qb[:, sl], kb[:, sl],
            (((1,), (1,)), ((), ())),
            preferred_element_type=jnp.float32)
        s = s + bias
        m = jnp.max(s, axis=-1, keepdims=True)
        p = jnp.exp(s - m)
        denom = jnp.sum(p, axis=-1, keepdims=True)
        recip = pl.reciprocal(denom, approx=True).reshape(1, -1)
        ctx_t = jax.lax.dot_general(
            vb[:, sl], p.astype(jnp.bfloat16),
            (((0,), (1,)), ((), ())),
            preferred_element_type=jnp.float32)
        ctx_t_parts.append((ctx_t * recip).astype(jnp.bfloat16))
    ctx_t = jnp.concatenate(ctx_t_parts, axis=0)

    attn = jax.lax.dot_general(
        ctx_t, wo_s[...],
        (((0,), (0,)), ((), ())),
        preferred_element_type=jnp.float32) + bo_ref[...]
    x1 = attn + xn

    x2 = _rms(x1, g2_ref[...], eps)

    n_a_mid = min(n_a_head + slots_a, len(chunks_a))

    @pl.when(first_step)
    def _load_w1():
        drain_b(0, len(chunks_b))
        drain_a(n_a_head, n_a_mid, len(chunks_a))

    h1 = jnp.dot(x2.astype(jnp.bfloat16), w1_s[...],
                 preferred_element_type=jnp.float32) + b1_ref[...]
    h1 = h1 * jax.nn.sigmoid(h1)

    @pl.when(first_step)
    def _load_w2():
        drain_a(n_a_mid, len(chunks_a), len(chunks_a))

    ff = jnp.dot(h1.astype(jnp.bfloat16), w2_s[...],
                 preferred_element_type=jnp.float32) + b2_ref[...]

    out_ref[...] = (ff + x2).astype(out_ref.dtype)


def kernel(latents, embeddings, mask, g1, wq, bq, wk, bk, wv, bv, wo, bo,
           g2, w1, b1, w2, b2):
    B, L, D = latents.shape
    _, S, _ = embeddings.shape
    H = 8
    hd = D // H
    HID = w1.shape[-1]
    scale = 1.0 / float(hd) ** 0.5

    if mask is None:
        mask = jnp.ones((B, S), dtype=jnp.float32)
    mask = mask.astype(jnp.float32)

    def row(v):
        return jnp.asarray(v).reshape(1, -1).astype(jnp.float32)

    f32 = jnp.float32
    params = [
        row(g1),
        wq.astype(f32), row(bq),
        wk.astype(f32), row(bk),
        wv.astype(f32), row(bv),
        wo.astype(f32), row(bo),
        row(g2),
        w1.astype(f32), row(b1),
        w2.astype(f32), row(b2),
    ]
    hbm_idx = {1, 3, 5, 7, 10, 12}

    buffered = getattr(pl, "Buffered", None)

    def build(single_buffer_weights):
        wkw = {"pipeline_mode": buffered(1)} if single_buffer_weights else {}

        def spec_for(i, arr):
            if i in hbm_idx:
                return pl.BlockSpec(memory_space=pl.ANY)
            nd = arr.ndim
            return pl.BlockSpec(arr.shape, lambda b, _nd=nd: (0,) * _nd, **wkw)

        in_specs = [
            pl.BlockSpec((None, L, D), lambda b: (b, 0, 0)),
            pl.BlockSpec((None, S, D), lambda b: (b, 0, 0)),
            pl.BlockSpec((B, S), lambda b: (0, 0), **wkw),
        ] + [spec_for(i, w) for i, w in enumerate(params)]

        return pl.pallas_call(
            functools.partial(_fused_kernel, num_heads=H, q_scale=scale),
            out_shape=jax.ShapeDtypeStruct((B, L, D), latents.dtype),
            grid=(B,),
            in_specs=in_specs,
            out_specs=pl.BlockSpec((None, L, D), lambda b: (b, 0, 0)),
            scratch_shapes=[
                pltpu.VMEM((D, D), jnp.bfloat16),
                pltpu.VMEM((D, D), jnp.bfloat16),
                pltpu.VMEM((D, D), jnp.bfloat16),
                pltpu.VMEM((D, D), jnp.bfloat16),
                pltpu.VMEM((D, HID), jnp.bfloat16),
                pltpu.VMEM((HID, D), jnp.bfloat16),
                pltpu.VMEM((4, min(512, D), D), jnp.float32),
                pltpu.VMEM((3, min(128, D), HID), jnp.float32),
                pltpu.SemaphoreType.DMA((4,)),
                pltpu.SemaphoreType.DMA((3,)),
            ],
            compiler_params=pltpu.CompilerParams(
                dimension_semantics=("arbitrary",),
                vmem_limit_bytes=(64 * 1024 * 1024 * 15) // 16,
            ),
        )

    args = (latents.astype(jnp.float32), embeddings.astype(jnp.float32),
            mask, *params)
    if buffered is not None:
        try:
            return build(True)(*args)
        except Exception:
            return build(False)(*args)
    return build(False)(*args)
